# baseline (speedup 1.0000x reference)
import functools
import math

import jax
import jax.numpy as jnp
import numpy as np
from jax import lax
from jax.experimental import pallas as pl
from jax.experimental.pallas import tpu as pltpu
from jax.experimental.pallas import tpu_sc as plsc

_T = 2048
_PRED = 256
_K = 16
_M = 1024
_TH = 1032
_R = 512


def _basis():
    tau = np.arange(_T, dtype=np.int64)[:, None]
    m = np.arange(1, _M + 1, dtype=np.int64)[None, :]
    ang = 2.0 * np.pi * ((tau * m) % _T).astype(np.float64) / _T
    c = np.cos(ang)
    s = np.sin(ang)
    c[:, -1] = 0.0
    s[:, -1] = 0.0
    return c.astype(np.float32), s.astype(np.float32)


_C_TABLE, _S_TABLE = _basis()


def _fwd_body(x_ref, xf_ref, c_ref, s_ref, re_ref, im_ref):
    xb = jnp.concatenate([x_ref[0], x_ref[1]], axis=1)
    xfb = jnp.concatenate([xf_ref[0], xf_ref[1]], axis=1)
    cb = c_ref[...]
    sb = s_ref[...]
    n = xb.shape[1]
    xr = xfb[0:1023, :]
    xmid = xb[1:1024, :]
    zpad = jnp.zeros((7, n), jnp.float32)
    zrow = jnp.zeros((1, n), jnp.float32)
    xe = jnp.concatenate([xb[0:1, :], xmid + xr, xb[1024:1025, :], zpad], 0)
    xo = jnp.concatenate([zrow, xmid - xr, zrow, zpad], 0)
    hi = jax.lax.Precision.HIGHEST
    dn = (((0,), (0,)), ((), ()))
    re_ref[...] = lax.dot_general(xe, cb, dn, precision=hi,
                                  preferred_element_type=jnp.float32)
    im_ref[...] = -lax.dot_general(xo, sb, dn, precision=hi,
                                   preferred_element_type=jnp.float32)


def _sc_body(re_hbm, im_hbm, a_hbm, b_hbm, re_v, im_v, av, bv):
    wid = lax.axis_index("s") * 2 + lax.axis_index("c")
    z16 = jnp.zeros((16,), jnp.float32)
    iota16 = lax.iota(jnp.int32, 16)

    def zero_buf(j, carry):
        av[pl.ds(j * 16, 16)] = z16
        bv[pl.ds(j * 16, 16)] = z16
        return carry

    lax.fori_loop(0, _M // 16, zero_buf, 0)

    def do_row(r, carry):
        row = wid * (_R // 32) + r
        pltpu.sync_copy(re_hbm.at[row], re_v)
        pltpu.sync_copy(im_hbm.at[row], im_v)

        def chunk(i, carry):
            best, bidx = carry
            vr = re_v[pl.ds(i * 16, 16)]
            vi = im_v[pl.ds(i * 16, 16)]
            vals = vr * vr + vi * vi
            cidx = iota16 + i * 16
            vd, idxd = plsc.sort_key_val(vals, cidx, descending=True)
            take = vd > best
            nb = jnp.where(take, vd, best)
            ni = jnp.where(take, idxd, bidx)
            nb2, ni2 = plsc.sort_key_val(nb, ni, descending=False)
            return (nb2, ni2)

        vr0 = re_v[pl.ds(0, 16)]
        vi0 = im_v[pl.ds(0, 16)]
        b0, i0 = plsc.sort_key_val(vr0 * vr0 + vi0 * vi0, iota16,
                                   descending=False)
        best, bidx = lax.fori_loop(1, _M // 16, chunk, (b0, i0))

        scale = 2.0 / _T
        va = plsc.load_gather(re_v, [bidx]) * scale
        vb = plsc.load_gather(im_v, [bidx]) * scale
        plsc.store_scatter(av, [bidx], va)
        plsc.store_scatter(bv, [bidx], vb)
        pltpu.sync_copy(av, a_hbm.at[row])
        pltpu.sync_copy(bv, b_hbm.at[row])
        plsc.store_scatter(av, [bidx], z16)
        plsc.store_scatter(bv, [bidx], z16)
        return carry

    lax.fori_loop(0, _R // 32, do_row, 0)


_sc_topk = functools.partial(
    pl.kernel,
    mesh=plsc.VectorSubcoreMesh(core_axis_name="c", subcore_axis_name="s"),
    out_type=(
        jax.ShapeDtypeStruct((_R, _M), jnp.float32),
        jax.ShapeDtypeStruct((_R, _M), jnp.float32),
    ),
    scratch_types=[
        pltpu.VMEM((_M,), jnp.float32),
        pltpu.VMEM((_M,), jnp.float32),
        pltpu.VMEM((_M,), jnp.float32),
        pltpu.VMEM((_M,), jnp.float32),
    ],
    compiler_params=pltpu.CompilerParams(needs_layout_passes=False),
)(_sc_body)


def _inv_body(a_ref, b_ref, c_ref, s_ref, o_ref):
    at = a_ref[...]
    bt = b_ref[...]
    cb = c_ref[...]
    sb = s_ref[...]
    dn = (((1,), (1,)), ((), ()))
    md = jax.lax.Precision.DEFAULT
    rec = (lax.dot_general(cb, at, dn, precision=md,
                           preferred_element_type=jnp.float32)
           - lax.dot_general(sb, bt, dn, precision=md,
                             preferred_element_type=jnp.float32))
    o_ref[0, : _T, :] = rec[:, :64]
    o_ref[0, _T:, :] = rec[: _PRED, :64]
    o_ref[1, : _T, :] = rec[:, 64:]
    o_ref[1, _T:, :] = rec[: _PRED, 64:]


@jax.jit
def kernel(x):
    b, t, d = x.shape
    xf = jnp.flip(x, axis=1)[:, : _T // 2, :]
    ch = jnp.asarray(_C_TABLE[:_TH])
    sh = jnp.asarray(_S_TABLE[:_TH])
    re_t, im_t = pl.pallas_call(
        _fwd_body,
        grid=(b // 2,),
        in_specs=[
            pl.BlockSpec((2, _TH, 64), lambda i: (i, 0, 0)),
            pl.BlockSpec((2, _T // 2, 64), lambda i: (i, 0, 0)),
            pl.BlockSpec((_TH, _M), lambda i: (0, 0)),
            pl.BlockSpec((_TH, _M), lambda i: (0, 0)),
        ],
        out_specs=[
            pl.BlockSpec((128, _M), lambda i: (i, 0)),
            pl.BlockSpec((128, _M), lambda i: (i, 0)),
        ],
        out_shape=[
            jax.ShapeDtypeStruct((_R, _M), jnp.float32),
            jax.ShapeDtypeStruct((_R, _M), jnp.float32),
        ],
    )(x, xf, ch, sh)
    a_t, b_t = _sc_topk(re_t, im_t)
    return pl.pallas_call(
        _inv_body,
        grid=(b // 2,),
        in_specs=[
            pl.BlockSpec((128, _M), lambda i: (i, 0)),
            pl.BlockSpec((128, _M), lambda i: (i, 0)),
            pl.BlockSpec((_T, _M), lambda i: (0, 0)),
            pl.BlockSpec((_T, _M), lambda i: (0, 0)),
        ],
        out_specs=pl.BlockSpec((2, _T + _PRED, 64), lambda i: (i, 0, 0)),
        out_shape=jax.ShapeDtypeStruct((b, t + _PRED, d), jnp.float32),
    )(a_t, b_t, jnp.asarray(_C_TABLE), jnp.asarray(_S_TABLE))


# speedup vs baseline: 1.2188x; 1.2188x over previous
import functools
import math

import jax
import jax.numpy as jnp
import numpy as np
from jax import lax
from jax.experimental import pallas as pl
from jax.experimental.pallas import tpu as pltpu
from jax.experimental.pallas import tpu_sc as plsc

_T = 2048
_PRED = 256
_K = 16
_M = 1024
_TH = 1032
_R = 512


def _basis():
    tau = np.arange(_T, dtype=np.int64)[:, None]
    m = np.arange(1, _M + 1, dtype=np.int64)[None, :]
    ang = 2.0 * np.pi * ((tau * m) % _T).astype(np.float64) / _T
    c = np.cos(ang)
    s = np.sin(ang)
    c[:, -1] = 0.0
    s[:, -1] = 0.0
    return c.astype(np.float32), s.astype(np.float32)


_C_TABLE, _S_TABLE = _basis()


def _fwd_body(x_ref, xf_ref, c_ref, s_ref, re_ref, im_ref):
    xb = jnp.concatenate([x_ref[0], x_ref[1]], axis=1)
    xfb = jnp.concatenate([xf_ref[0], xf_ref[1]], axis=1)
    cb = c_ref[...]
    sb = s_ref[...]
    n = xb.shape[1]
    xr = xfb[0:1023, :]
    xmid = xb[1:1024, :]
    zpad = jnp.zeros((7, n), jnp.float32)
    zrow = jnp.zeros((1, n), jnp.float32)
    xe = jnp.concatenate([xb[0:1, :], xmid + xr, xb[1024:1025, :], zpad], 0)
    xo = jnp.concatenate([zrow, xmid - xr, zrow, zpad], 0)
    hi = jax.lax.Precision.HIGHEST
    dn = (((0,), (0,)), ((), ()))
    re_ref[...] = lax.dot_general(xe, cb, dn, precision=hi,
                                  preferred_element_type=jnp.float32)
    im_ref[...] = -lax.dot_general(xo, sb, dn, precision=hi,
                                   preferred_element_type=jnp.float32)


_RPW = _R // 32
_NCH = 4


def _sc_body(re_hbm, im_hbm, a_hbm, b_hbm, re_v, im_v, av, bv):
    wid = lax.axis_index("s") * 2 + lax.axis_index("c")
    base = wid * _RPW
    z16 = jnp.zeros((16,), jnp.float32)
    iota16 = lax.iota(jnp.int32, 16)

    pltpu.sync_copy(re_hbm.at[pl.ds(base, _RPW)], re_v)
    pltpu.sync_copy(im_hbm.at[pl.ds(base, _RPW)], im_v)

    def zero_buf(j, carry):
        for r in range(_RPW):
            av[r, pl.ds(j * 16, 16)] = z16
            bv[r, pl.ds(j * 16, 16)] = z16
        return carry

    lax.fori_loop(0, _M // 16, zero_buf, 0)

    nper = _M // 16 // _NCH

    def do_row(r, carry):
        def mag(i):
            vr = re_v[r, pl.ds(i * 16, 16)]
            vi = im_v[r, pl.ds(i * 16, 16)]
            return vr * vr + vi * vi

        chains = []
        for k in range(_NCH):
            b0, i0 = plsc.sort_key_val(mag(k * nper), iota16 + k * nper * 16,
                                       descending=False)
            chains.extend([b0, i0])

        def chunk(i, carry):
            nxt = []
            for k in range(_NCH):
                best, bidx = carry[2 * k], carry[2 * k + 1]
                ic = k * nper + i
                vd, idxd = plsc.sort_key_val(mag(ic), iota16 + ic * 16,
                                             descending=True)
                take = vd > best
                nb = jnp.where(take, vd, best)
                ni = jnp.where(take, idxd, bidx)
                nb2, ni2 = plsc.sort_key_val(nb, ni, descending=False)
                nxt.extend([nb2, ni2])
            return tuple(nxt)

        chains = lax.fori_loop(1, nper, chunk, tuple(chains))

        def merge(b_a, i_a, b_b, i_b):
            bd, idd = plsc.sort_key_val(b_b, i_b, descending=True)
            take = bd > b_a
            nb = jnp.where(take, bd, b_a)
            ni = jnp.where(take, idd, i_a)
            nb2, ni2 = plsc.sort_key_val(nb, ni, descending=False)
            return nb2, ni2

        b01, i01 = merge(chains[0], chains[1], chains[2], chains[3])
        b23, i23 = merge(chains[4], chains[5], chains[6], chains[7])
        best, bidx = merge(b01, i01, b23, i23)

        rv = iota16 * 0 + r
        scale = 2.0 / _T
        va = plsc.load_gather(re_v, [rv, bidx]) * scale
        vb = plsc.load_gather(im_v, [rv, bidx]) * scale
        plsc.store_scatter(av, [rv, bidx], va)
        plsc.store_scatter(bv, [rv, bidx], vb)
        return carry

    lax.fori_loop(0, _RPW, do_row, 0)
    pltpu.sync_copy(av, a_hbm.at[pl.ds(base, _RPW)])
    pltpu.sync_copy(bv, b_hbm.at[pl.ds(base, _RPW)])


_sc_topk = functools.partial(
    pl.kernel,
    mesh=plsc.VectorSubcoreMesh(core_axis_name="c", subcore_axis_name="s"),
    out_type=(
        jax.ShapeDtypeStruct((_R, _M), jnp.float32),
        jax.ShapeDtypeStruct((_R, _M), jnp.float32),
    ),
    scratch_types=[
        pltpu.VMEM((_RPW, _M), jnp.float32),
        pltpu.VMEM((_RPW, _M), jnp.float32),
        pltpu.VMEM((_RPW, _M), jnp.float32),
        pltpu.VMEM((_RPW, _M), jnp.float32),
    ],
    compiler_params=pltpu.CompilerParams(needs_layout_passes=False),
)(_sc_body)


def _inv_body(a_ref, b_ref, c_ref, s_ref, o_ref):
    at = a_ref[...]
    bt = b_ref[...]
    cb = c_ref[...]
    sb = s_ref[...]
    dn = (((1,), (1,)), ((), ()))
    md = jax.lax.Precision.DEFAULT
    rec = (lax.dot_general(cb, at, dn, precision=md,
                           preferred_element_type=jnp.float32)
           - lax.dot_general(sb, bt, dn, precision=md,
                             preferred_element_type=jnp.float32))
    o_ref[0, : _T, :] = rec[:, :64]
    o_ref[0, _T:, :] = rec[: _PRED, :64]
    o_ref[1, : _T, :] = rec[:, 64:]
    o_ref[1, _T:, :] = rec[: _PRED, 64:]


@jax.jit
def kernel(x):
    b, t, d = x.shape
    xf = jnp.flip(x, axis=1)[:, : _T // 2, :]
    ch = jnp.asarray(_C_TABLE[:_TH])
    sh = jnp.asarray(_S_TABLE[:_TH])
    re_t, im_t = pl.pallas_call(
        _fwd_body,
        grid=(b // 2,),
        in_specs=[
            pl.BlockSpec((2, _TH, 64), lambda i: (i, 0, 0)),
            pl.BlockSpec((2, _T // 2, 64), lambda i: (i, 0, 0)),
            pl.BlockSpec((_TH, _M), lambda i: (0, 0)),
            pl.BlockSpec((_TH, _M), lambda i: (0, 0)),
        ],
        out_specs=[
            pl.BlockSpec((128, _M), lambda i: (i, 0)),
            pl.BlockSpec((128, _M), lambda i: (i, 0)),
        ],
        out_shape=[
            jax.ShapeDtypeStruct((_R, _M), jnp.float32),
            jax.ShapeDtypeStruct((_R, _M), jnp.float32),
        ],
    )(x, xf, ch, sh)
    a_t, b_t = _sc_topk(re_t, im_t)
    return pl.pallas_call(
        _inv_body,
        grid=(b // 2,),
        in_specs=[
            pl.BlockSpec((128, _M), lambda i: (i, 0)),
            pl.BlockSpec((128, _M), lambda i: (i, 0)),
            pl.BlockSpec((_T, _M), lambda i: (0, 0)),
            pl.BlockSpec((_T, _M), lambda i: (0, 0)),
        ],
        out_specs=pl.BlockSpec((2, _T + _PRED, 64), lambda i: (i, 0, 0)),
        out_shape=jax.ShapeDtypeStruct((b, t + _PRED, d), jnp.float32),
    )(a_t, b_t, jnp.asarray(_C_TABLE), jnp.asarray(_S_TABLE))


# speedup vs baseline: 1.2448x; 1.0213x over previous
import functools
import math

import jax
import jax.numpy as jnp
import ml_dtypes
import numpy as np
from jax import lax
from jax.experimental import pallas as pl
from jax.experimental.pallas import tpu as pltpu
from jax.experimental.pallas import tpu_sc as plsc

_T = 2048
_PRED = 256
_K = 16
_M = 1024
_TH = 1032
_R = 512


def _basis():
    tau = np.arange(_T, dtype=np.int64)[:, None]
    m = np.arange(1, _M + 1, dtype=np.int64)[None, :]
    ang = 2.0 * np.pi * ((tau * m) % _T).astype(np.float64) / _T
    c = np.cos(ang)
    s = np.sin(ang)
    c[:, -1] = 0.0
    s[:, -1] = 0.0
    return c.astype(np.float32), s.astype(np.float32)


_C_TABLE, _S_TABLE = _basis()
_C16 = _C_TABLE.astype(ml_dtypes.bfloat16)
_S16 = _S_TABLE.astype(ml_dtypes.bfloat16)


def _fwd_body(x_ref, xf_ref, c_ref, s_ref, re_ref, im_ref):
    xb = jnp.concatenate([x_ref[0], x_ref[1]], axis=1)
    xfb = jnp.concatenate([xf_ref[0], xf_ref[1]], axis=1)
    cb = c_ref[...]
    sb = s_ref[...]
    n = xb.shape[1]
    xr = xfb[0:1023, :]
    xmid = xb[1:1024, :]
    zpad = jnp.zeros((7, n), jnp.float32)
    zrow = jnp.zeros((1, n), jnp.float32)
    xe = jnp.concatenate([xb[0:1, :], xmid + xr, xb[1024:1025, :], zpad], 0)
    xo = jnp.concatenate([zrow, xmid - xr, zrow, zpad], 0)
    hi = jax.lax.Precision.HIGHEST
    dn = (((0,), (0,)), ((), ()))
    re_ref[...] = lax.dot_general(xe, cb, dn, precision=hi,
                                  preferred_element_type=jnp.float32)
    im_ref[...] = -lax.dot_general(xo, sb, dn, precision=hi,
                                   preferred_element_type=jnp.float32)


_RPW = _R // 32
_NCH = 4


def _sc_body(re_hbm, im_hbm, a_hbm, b_hbm, re_v, im_v, av, bv):
    wid = lax.axis_index("s") * 2 + lax.axis_index("c")
    base = wid * _RPW
    z16 = jnp.zeros((16,), jnp.float32)
    iota16 = lax.iota(jnp.int32, 16)

    pltpu.sync_copy(re_hbm.at[pl.ds(base, _RPW)], re_v)
    pltpu.sync_copy(im_hbm.at[pl.ds(base, _RPW)], im_v)

    def zero_buf(j, carry):
        for r in range(_RPW):
            av[r, pl.ds(j * 16, 16)] = z16
            bv[r, pl.ds(j * 16, 16)] = z16
        return carry

    lax.fori_loop(0, _M // 16, zero_buf, 0)

    nper = _M // 16 // _NCH

    def do_row(r, carry):
        def mag(i):
            vr = re_v[r, pl.ds(i * 16, 16)]
            vi = im_v[r, pl.ds(i * 16, 16)]
            return vr * vr + vi * vi

        chains = []
        for k in range(_NCH):
            b0, i0 = plsc.sort_key_val(mag(k * nper), iota16 + k * nper * 16,
                                       descending=False)
            chains.extend([b0, i0])

        def chunk(i, carry):
            nxt = []
            for k in range(_NCH):
                best, bidx = carry[2 * k], carry[2 * k + 1]
                ic = k * nper + i
                vd, idxd = plsc.sort_key_val(mag(ic), iota16 + ic * 16,
                                             descending=True)
                take = vd > best
                nb = jnp.where(take, vd, best)
                ni = jnp.where(take, idxd, bidx)
                nb2, ni2 = plsc.sort_key_val(nb, ni, descending=False)
                nxt.extend([nb2, ni2])
            return tuple(nxt)

        chains = lax.fori_loop(1, nper, chunk, tuple(chains))

        def merge(b_a, i_a, b_b, i_b):
            bd, idd = plsc.sort_key_val(b_b, i_b, descending=True)
            take = bd > b_a
            nb = jnp.where(take, bd, b_a)
            ni = jnp.where(take, idd, i_a)
            nb2, ni2 = plsc.sort_key_val(nb, ni, descending=False)
            return nb2, ni2

        b01, i01 = merge(chains[0], chains[1], chains[2], chains[3])
        b23, i23 = merge(chains[4], chains[5], chains[6], chains[7])
        best, bidx = merge(b01, i01, b23, i23)

        rv = iota16 * 0 + r
        scale = 2.0 / _T
        va = plsc.load_gather(re_v, [rv, bidx]) * scale
        vb = plsc.load_gather(im_v, [rv, bidx]) * scale
        plsc.store_scatter(av, [rv, bidx], va)
        plsc.store_scatter(bv, [rv, bidx], vb)
        return carry

    lax.fori_loop(0, _RPW, do_row, 0)
    pltpu.sync_copy(av, a_hbm.at[pl.ds(base, _RPW)])
    pltpu.sync_copy(bv, b_hbm.at[pl.ds(base, _RPW)])


_sc_topk = functools.partial(
    pl.kernel,
    mesh=plsc.VectorSubcoreMesh(core_axis_name="c", subcore_axis_name="s"),
    out_type=(
        jax.ShapeDtypeStruct((_R, _M), jnp.float32),
        jax.ShapeDtypeStruct((_R, _M), jnp.float32),
    ),
    scratch_types=[
        pltpu.VMEM((_RPW, _M), jnp.float32),
        pltpu.VMEM((_RPW, _M), jnp.float32),
        pltpu.VMEM((_RPW, _M), jnp.float32),
        pltpu.VMEM((_RPW, _M), jnp.float32),
    ],
    compiler_params=pltpu.CompilerParams(needs_layout_passes=False),
)(_sc_body)


def _inv_body(a_ref, b_ref, c_ref, s_ref, o_ref):
    at = a_ref[...].astype(jnp.bfloat16)
    bt = b_ref[...].astype(jnp.bfloat16)
    cb = c_ref[...]
    sb = s_ref[...]
    dn = (((1,), (1,)), ((), ()))
    md = jax.lax.Precision.DEFAULT
    rec = (lax.dot_general(cb, at, dn, precision=md,
                           preferred_element_type=jnp.float32)
           - lax.dot_general(sb, bt, dn, precision=md,
                             preferred_element_type=jnp.float32))
    o_ref[0, : _T, :] = rec[:, :64]
    o_ref[0, _T:, :] = rec[: _PRED, :64]
    o_ref[1, : _T, :] = rec[:, 64:]
    o_ref[1, _T:, :] = rec[: _PRED, 64:]


@jax.jit
def kernel(x):
    b, t, d = x.shape
    xf = jnp.flip(x, axis=1)[:, : _T // 2, :]
    ch = jnp.asarray(_C_TABLE[:_TH])
    sh = jnp.asarray(_S_TABLE[:_TH])
    re_t, im_t = pl.pallas_call(
        _fwd_body,
        grid=(b // 2,),
        in_specs=[
            pl.BlockSpec((2, _TH, 64), lambda i: (i, 0, 0)),
            pl.BlockSpec((2, _T // 2, 64), lambda i: (i, 0, 0)),
            pl.BlockSpec((_TH, _M), lambda i: (0, 0)),
            pl.BlockSpec((_TH, _M), lambda i: (0, 0)),
        ],
        out_specs=[
            pl.BlockSpec((128, _M), lambda i: (i, 0)),
            pl.BlockSpec((128, _M), lambda i: (i, 0)),
        ],
        out_shape=[
            jax.ShapeDtypeStruct((_R, _M), jnp.float32),
            jax.ShapeDtypeStruct((_R, _M), jnp.float32),
        ],
    )(x, xf, ch, sh)
    a_t, b_t = _sc_topk(re_t, im_t)
    return pl.pallas_call(
        _inv_body,
        grid=(b // 2,),
        in_specs=[
            pl.BlockSpec((128, _M), lambda i: (i, 0)),
            pl.BlockSpec((128, _M), lambda i: (i, 0)),
            pl.BlockSpec((_T, _M), lambda i: (0, 0)),
            pl.BlockSpec((_T, _M), lambda i: (0, 0)),
        ],
        out_specs=pl.BlockSpec((2, _T + _PRED, 64), lambda i: (i, 0, 0)),
        out_shape=jax.ShapeDtypeStruct((b, t + _PRED, d), jnp.float32),
    )(a_t, b_t, jnp.asarray(_C16), jnp.asarray(_S16))


# speedup vs baseline: 1.3097x; 1.0522x over previous
import functools
import math

import jax
import jax.numpy as jnp
import ml_dtypes
import numpy as np
from jax import lax
from jax.experimental import pallas as pl
from jax.experimental.pallas import tpu as pltpu
from jax.experimental.pallas import tpu_sc as plsc

_T = 2048
_PRED = 256
_K = 16
_M = 1024
_TH = 1032
_R = 512


def _basis():
    tau = np.arange(_T, dtype=np.int64)[:, None]
    m = np.arange(1, _M + 1, dtype=np.int64)[None, :]
    ang = 2.0 * np.pi * ((tau * m) % _T).astype(np.float64) / _T
    c = np.cos(ang)
    s = np.sin(ang)
    c[:, -1] = 0.0
    s[:, -1] = 0.0
    return c.astype(np.float32), s.astype(np.float32)


_C_TABLE, _S_TABLE = _basis()
_C16 = _C_TABLE.astype(ml_dtypes.bfloat16)
_S16 = _S_TABLE.astype(ml_dtypes.bfloat16)


def _fwd_body(x_ref, xf_ref, c_ref, s_ref, re_ref, im_ref):
    xb = jnp.concatenate([x_ref[0], x_ref[1]], axis=1)
    xfb = jnp.concatenate([xf_ref[0], xf_ref[1]], axis=1)
    cb = c_ref[...]
    sb = s_ref[...]
    n = xb.shape[1]
    xr = xfb[0:1023, :]
    xmid = xb[1:1024, :]
    zpad = jnp.zeros((7, n), jnp.float32)
    zrow = jnp.zeros((1, n), jnp.float32)
    xe = jnp.concatenate([xb[0:1, :], xmid + xr, xb[1024:1025, :], zpad], 0)
    xo = jnp.concatenate([zrow, xmid - xr, zrow, zpad], 0)
    hi = jax.lax.Precision.HIGHEST
    dn = (((0,), (0,)), ((), ()))
    re_ref[...] = lax.dot_general(xe, cb, dn, precision=hi,
                                  preferred_element_type=jnp.float32)
    im_ref[...] = -lax.dot_general(xo, sb, dn, precision=hi,
                                   preferred_element_type=jnp.float32)


_RPW = _R // 32
_NCH = 4


def _sc_body(re_hbm, im_hbm, a_hbm, b_hbm, re_v, im_v, av, bv, sem1, sem2):
    wid = lax.axis_index("s") * 2 + lax.axis_index("c")
    base = wid * _RPW
    z16 = jnp.zeros((16,), jnp.float32)
    iota16 = lax.iota(jnp.int32, 16)

    cp1 = pltpu.async_copy(re_hbm.at[pl.ds(base, _RPW)], re_v, sem1)
    cp2 = pltpu.async_copy(im_hbm.at[pl.ds(base, _RPW)], im_v, sem2)

    def zero_buf(j, carry):
        for r in range(_RPW):
            av[r, pl.ds(j * 16, 16)] = z16
            bv[r, pl.ds(j * 16, 16)] = z16
        return carry

    lax.fori_loop(0, _M // 16, zero_buf, 0)
    cp1.wait()
    cp2.wait()

    nper = _M // 16 // _NCH

    def do_row(r, carry):
        def mag(i):
            vr = re_v[r, pl.ds(i * 16, 16)]
            vi = im_v[r, pl.ds(i * 16, 16)]
            return vr * vr + vi * vi

        chains = []
        for k in range(_NCH):
            b0, i0 = plsc.sort_key_val(mag(k * nper), iota16 + k * nper * 16,
                                       descending=False)
            chains.extend([b0, i0])

        def chunk(i, carry):
            nxt = []
            for k in range(_NCH):
                best, bidx = carry[2 * k], carry[2 * k + 1]
                ic = k * nper + i
                vd, idxd = plsc.sort_key_val(mag(ic), iota16 + ic * 16,
                                             descending=True)
                take = vd > best
                nb = jnp.where(take, vd, best)
                ni = jnp.where(take, idxd, bidx)
                nb2, ni2 = plsc.sort_key_val(nb, ni, descending=False)
                nxt.extend([nb2, ni2])
            return tuple(nxt)

        chains = lax.fori_loop(1, nper, chunk, tuple(chains))

        def merge(b_a, i_a, b_b, i_b):
            bd, idd = plsc.sort_key_val(b_b, i_b, descending=True)
            take = bd > b_a
            nb = jnp.where(take, bd, b_a)
            ni = jnp.where(take, idd, i_a)
            nb2, ni2 = plsc.sort_key_val(nb, ni, descending=False)
            return nb2, ni2

        b01, i01 = merge(chains[0], chains[1], chains[2], chains[3])
        b23, i23 = merge(chains[4], chains[5], chains[6], chains[7])
        best, bidx = merge(b01, i01, b23, i23)

        rv = iota16 * 0 + r
        scale = 2.0 / _T
        va = plsc.load_gather(re_v, [rv, bidx]) * scale
        vb = plsc.load_gather(im_v, [rv, bidx]) * scale
        plsc.store_scatter(av, [rv, bidx], va)
        plsc.store_scatter(bv, [rv, bidx], vb)
        return carry

    lax.fori_loop(0, _RPW, do_row, 0)
    pltpu.sync_copy(av, a_hbm.at[pl.ds(base, _RPW)])
    pltpu.sync_copy(bv, b_hbm.at[pl.ds(base, _RPW)])


_sc_topk = functools.partial(
    pl.kernel,
    mesh=plsc.VectorSubcoreMesh(core_axis_name="c", subcore_axis_name="s"),
    out_type=(
        jax.ShapeDtypeStruct((_R, _M), jnp.float32),
        jax.ShapeDtypeStruct((_R, _M), jnp.float32),
    ),
    scratch_types=[
        pltpu.VMEM((_RPW, _M), jnp.float32),
        pltpu.VMEM((_RPW, _M), jnp.float32),
        pltpu.VMEM((_RPW, _M), jnp.float32),
        pltpu.VMEM((_RPW, _M), jnp.float32),
        pltpu.SemaphoreType.DMA,
        pltpu.SemaphoreType.DMA,
    ],
    compiler_params=pltpu.CompilerParams(needs_layout_passes=False),
)(_sc_body)


def _inv_body(a_ref, b_ref, c_ref, s_ref, o_ref):
    at = a_ref[...].astype(jnp.bfloat16)
    bt = b_ref[...].astype(jnp.bfloat16)
    cb = c_ref[...]
    sb = s_ref[...]
    dn = (((1,), (1,)), ((), ()))
    md = jax.lax.Precision.DEFAULT
    rec = (lax.dot_general(cb, at, dn, precision=md,
                           preferred_element_type=jnp.float32)
           - lax.dot_general(sb, bt, dn, precision=md,
                             preferred_element_type=jnp.float32))
    for k in range(4):
        cols = slice(64 * k, 64 * (k + 1))
        o_ref[k, : _T, :] = rec[:, cols]
        o_ref[k, _T:, :] = rec[: _PRED, cols]


@jax.jit
def kernel(x):
    b, t, d = x.shape
    xf = jnp.flip(x, axis=1)[:, : _T // 2, :]
    ch = jnp.asarray(_C_TABLE[:_TH])
    sh = jnp.asarray(_S_TABLE[:_TH])
    re_t, im_t = pl.pallas_call(
        _fwd_body,
        grid=(b // 2,),
        in_specs=[
            pl.BlockSpec((2, _TH, 64), lambda i: (i, 0, 0)),
            pl.BlockSpec((2, _T // 2, 64), lambda i: (i, 0, 0)),
            pl.BlockSpec((_TH, _M), lambda i: (0, 0)),
            pl.BlockSpec((_TH, _M), lambda i: (0, 0)),
        ],
        out_specs=[
            pl.BlockSpec((128, _M), lambda i: (i, 0)),
            pl.BlockSpec((128, _M), lambda i: (i, 0)),
        ],
        out_shape=[
            jax.ShapeDtypeStruct((_R, _M), jnp.float32),
            jax.ShapeDtypeStruct((_R, _M), jnp.float32),
        ],
    )(x, xf, ch, sh)
    a_t, b_t = _sc_topk(re_t, im_t)
    return pl.pallas_call(
        _inv_body,
        grid=(b // 4,),
        in_specs=[
            pl.BlockSpec((256, _M), lambda i: (i, 0)),
            pl.BlockSpec((256, _M), lambda i: (i, 0)),
            pl.BlockSpec((_T, _M), lambda i: (0, 0)),
            pl.BlockSpec((_T, _M), lambda i: (0, 0)),
        ],
        out_specs=pl.BlockSpec((4, _T + _PRED, 64), lambda i: (i, 0, 0)),
        out_shape=jax.ShapeDtypeStruct((b, t + _PRED, d), jnp.float32),
    )(a_t, b_t, jnp.asarray(_C16), jnp.asarray(_S16))


# speedup vs baseline: 1.3419x; 1.0246x over previous
import functools
import math

import jax
import jax.numpy as jnp
import ml_dtypes
import numpy as np
from jax import lax
from jax.experimental import pallas as pl
from jax.experimental.pallas import tpu as pltpu
from jax.experimental.pallas import tpu_sc as plsc

_T = 2048
_PRED = 256
_K = 16
_M = 1024
_TH = 1032
_R = 512


def _basis():
    tau = np.arange(_T, dtype=np.int64)[:, None]
    m = np.arange(1, _M + 1, dtype=np.int64)[None, :]
    ang = 2.0 * np.pi * ((tau * m) % _T).astype(np.float64) / _T
    c = np.cos(ang)
    s = np.sin(ang)
    c[:, -1] = 0.0
    s[:, -1] = 0.0
    return c.astype(np.float32), s.astype(np.float32)


_C_TABLE, _S_TABLE = _basis()
_C16 = _C_TABLE.astype(ml_dtypes.bfloat16)
_S16 = _S_TABLE.astype(ml_dtypes.bfloat16)


def _fwd_body(x_ref, xf_ref, c_ref, s_ref, re_ref, im_ref):
    xb = jnp.concatenate([x_ref[k] for k in range(4)], axis=1)
    xfb = jnp.concatenate([xf_ref[k] for k in range(4)], axis=1)
    cb = c_ref[...]
    sb = s_ref[...]
    n = xb.shape[1]
    xr = xfb[0:1023, :]
    xmid = xb[1:1024, :]
    zpad = jnp.zeros((7, n), jnp.float32)
    zrow = jnp.zeros((1, n), jnp.float32)
    xe = jnp.concatenate([xb[0:1, :], xmid + xr, xb[1024:1025, :], zpad], 0)
    xo = jnp.concatenate([zrow, xmid - xr, zrow, zpad], 0)
    hi = jax.lax.Precision.HIGHEST
    dn = (((0,), (0,)), ((), ()))
    re_ref[...] = lax.dot_general(xe, cb, dn, precision=hi,
                                  preferred_element_type=jnp.float32)
    im_ref[...] = -lax.dot_general(xo, sb, dn, precision=hi,
                                   preferred_element_type=jnp.float32)


_RPW = _R // 32
_NCH = 4


def _sc_body(re_hbm, im_hbm, a_hbm, b_hbm, re_v, im_v, av, bv, sem1, sem2):
    wid = lax.axis_index("s") * 2 + lax.axis_index("c")
    base = wid * _RPW
    z16 = jnp.zeros((16,), jnp.float32)
    iota16 = lax.iota(jnp.int32, 16)

    cp1 = pltpu.async_copy(re_hbm.at[pl.ds(base, _RPW)], re_v, sem1)
    cp2 = pltpu.async_copy(im_hbm.at[pl.ds(base, _RPW)], im_v, sem2)

    def zero_buf(j, carry):
        for r in range(_RPW):
            av[r, pl.ds(j * 16, 16)] = z16
            bv[r, pl.ds(j * 16, 16)] = z16
        return carry

    lax.fori_loop(0, _M // 16, zero_buf, 0)
    cp1.wait()
    cp2.wait()

    nper = _M // 16 // _NCH

    def do_row(r, carry):
        def mag(i):
            vr = re_v[r, pl.ds(i * 16, 16)]
            vi = im_v[r, pl.ds(i * 16, 16)]
            return vr * vr + vi * vi

        chains = []
        for k in range(_NCH):
            b0, i0 = plsc.sort_key_val(mag(k * nper), iota16 + k * nper * 16,
                                       descending=False)
            chains.extend([b0, i0])

        def chunk(i, carry):
            nxt = []
            for k in range(_NCH):
                best, bidx = carry[2 * k], carry[2 * k + 1]
                ic = k * nper + i
                vd, idxd = plsc.sort_key_val(mag(ic), iota16 + ic * 16,
                                             descending=True)
                take = vd > best
                nb = jnp.where(take, vd, best)
                ni = jnp.where(take, idxd, bidx)
                nb2, ni2 = plsc.sort_key_val(nb, ni, descending=False)
                nxt.extend([nb2, ni2])
            return tuple(nxt)

        chains = lax.fori_loop(1, nper, chunk, tuple(chains))

        def merge(b_a, i_a, b_b, i_b):
            bd, idd = plsc.sort_key_val(b_b, i_b, descending=True)
            take = bd > b_a
            nb = jnp.where(take, bd, b_a)
            ni = jnp.where(take, idd, i_a)
            nb2, ni2 = plsc.sort_key_val(nb, ni, descending=False)
            return nb2, ni2

        b01, i01 = merge(chains[0], chains[1], chains[2], chains[3])
        b23, i23 = merge(chains[4], chains[5], chains[6], chains[7])
        best, bidx = merge(b01, i01, b23, i23)

        rv = iota16 * 0 + r
        scale = 2.0 / _T
        va = plsc.load_gather(re_v, [rv, bidx]) * scale
        vb = plsc.load_gather(im_v, [rv, bidx]) * scale
        plsc.store_scatter(av, [rv, bidx], va)
        plsc.store_scatter(bv, [rv, bidx], vb)
        return carry

    lax.fori_loop(0, _RPW, do_row, 0)
    pltpu.sync_copy(av, a_hbm.at[pl.ds(base, _RPW)])
    pltpu.sync_copy(bv, b_hbm.at[pl.ds(base, _RPW)])


_sc_topk = functools.partial(
    pl.kernel,
    mesh=plsc.VectorSubcoreMesh(core_axis_name="c", subcore_axis_name="s"),
    out_type=(
        jax.ShapeDtypeStruct((_R, _M), jnp.float32),
        jax.ShapeDtypeStruct((_R, _M), jnp.float32),
    ),
    scratch_types=[
        pltpu.VMEM((_RPW, _M), jnp.float32),
        pltpu.VMEM((_RPW, _M), jnp.float32),
        pltpu.VMEM((_RPW, _M), jnp.float32),
        pltpu.VMEM((_RPW, _M), jnp.float32),
        pltpu.SemaphoreType.DMA,
        pltpu.SemaphoreType.DMA,
    ],
    compiler_params=pltpu.CompilerParams(needs_layout_passes=False),
)(_sc_body)


def _inv_body(a_ref, b_ref, c_ref, s_ref, o_ref):
    at = a_ref[...].astype(jnp.bfloat16)
    bt = b_ref[...].astype(jnp.bfloat16)
    cb = c_ref[...]
    sb = s_ref[...]
    dn = (((1,), (1,)), ((), ()))
    md = jax.lax.Precision.DEFAULT
    rec = (lax.dot_general(cb, at, dn, precision=md,
                           preferred_element_type=jnp.float32)
           - lax.dot_general(sb, bt, dn, precision=md,
                             preferred_element_type=jnp.float32))
    for k in range(4):
        cols = slice(64 * k, 64 * (k + 1))
        o_ref[k, : _T, :] = rec[:, cols]
        o_ref[k, _T:, :] = rec[: _PRED, cols]


@jax.jit
def kernel(x):
    b, t, d = x.shape
    xf = jnp.flip(x, axis=1)[:, : _T // 2, :]
    ch = jnp.asarray(_C_TABLE[:_TH])
    sh = jnp.asarray(_S_TABLE[:_TH])
    re_t, im_t = pl.pallas_call(
        _fwd_body,
        grid=(b // 4,),
        in_specs=[
            pl.BlockSpec((4, _TH, 64), lambda i: (i, 0, 0)),
            pl.BlockSpec((4, _T // 2, 64), lambda i: (i, 0, 0)),
            pl.BlockSpec((_TH, _M), lambda i: (0, 0)),
            pl.BlockSpec((_TH, _M), lambda i: (0, 0)),
        ],
        out_specs=[
            pl.BlockSpec((256, _M), lambda i: (i, 0)),
            pl.BlockSpec((256, _M), lambda i: (i, 0)),
        ],
        out_shape=[
            jax.ShapeDtypeStruct((_R, _M), jnp.float32),
            jax.ShapeDtypeStruct((_R, _M), jnp.float32),
        ],
    )(x, xf, ch, sh)
    a_t, b_t = _sc_topk(re_t, im_t)
    return pl.pallas_call(
        _inv_body,
        grid=(b // 4,),
        in_specs=[
            pl.BlockSpec((256, _M), lambda i: (i, 0)),
            pl.BlockSpec((256, _M), lambda i: (i, 0)),
            pl.BlockSpec((_T, _M), lambda i: (0, 0)),
            pl.BlockSpec((_T, _M), lambda i: (0, 0)),
        ],
        out_specs=pl.BlockSpec((4, _T + _PRED, 64), lambda i: (i, 0, 0)),
        out_shape=jax.ShapeDtypeStruct((b, t + _PRED, d), jnp.float32),
    )(a_t, b_t, jnp.asarray(_C16), jnp.asarray(_S16))
